# baseline (speedup 1.0000x reference)
import jax
import jax.numpy as jnp
from jax import lax
from jax.experimental import pallas as pl
from jax.experimental.pallas import tpu as pltpu
from jax.experimental.pallas import tpu_sc as plsc

N = 10000
E = 320000
D = 128
H = D // 2

NC = 2
NS = 16
B = 128
EP = 327680
EPT = EP // NS
NBLK = EPT // B
NP = 10240
RPS = NP // NS
ZR = 32

f32 = jnp.float32

_mesh = plsc.VectorSubcoreMesh(core_axis_name="c", subcore_axis_name="s")


def _zero_vmem(buf, rows, width):
    def zrow(i, carry):
        for j in range(width // 16):
            buf[i, pl.ds(j * 16, 16)] = jnp.zeros((16,), f32)
        return carry
    lax.fori_loop(0, rows, zrow, 0)


def _edge_loop(u_hbm, src_i, dst_i, bufs, agg_sh, sems_g, sems_s, lead,
               cnt_sh=None, ones_v=None, sem_c=None):
    nslot = len(bufs)
    for k in range(lead):
        pltpu.async_copy(u_hbm.at[src_i.at[k]], bufs[k], sems_g[k])

    dwait = nslot - lead

    def grp(g, carry):
        t0 = g * nslot
        for k in range(nslot):
            t = t0 + k
            buf, sem_g, sem_s = bufs[k], sems_g[k], sems_s[k]
            pltpu.make_async_copy(u_hbm.at[src_i.at[t]], buf, sem_g).wait()
            pltpu.async_copy(buf, agg_sh.at[dst_i.at[t]], sem_s, add=True)
            if cnt_sh is not None:
                pltpu.async_copy(ones_v, cnt_sh.at[dst_i.at[t]], sem_c,
                                 add=True)

            kw = (k - dwait) % nslot

            @pl.when(t >= dwait)
            def _():
                pltpu.make_async_copy(bufs[kw], agg_sh.at[dst_i.at[t]],
                                      sems_s[kw]).wait()

            kn = (k + lead) % nslot

            @pl.when(t + lead < NBLK)
            def _():
                pltpu.async_copy(u_hbm.at[src_i.at[t + lead]], bufs[kn],
                                 sems_g[kn])
        return carry
    lax.fori_loop(0, NBLK // nslot, grp, 0)

    for t in range(NBLK - dwait, NBLK):
        k = t % nslot
        pltpu.make_async_copy(bufs[k], agg_sh.at[dst_i.at[0]],
                              sems_s[k]).wait()

    if cnt_sh is not None:
        def drain(i, carry):
            pltpu.make_async_copy(ones_v, cnt_sh.at[dst_i.at[0]],
                                  sem_c).wait()
            return carry
        lax.fori_loop(0, NBLK, drain, 0)


def _sc_agg_cnt_body(ua_hbm, ub_hbm, src_hbm, dst_hbm, agg_out, cnt_out,
                     src_i, dst_i, b0, b1, b2, b3,
                     zbuf, zcnt, ones_v,
                     agg_sh, cnt_sh, g0, g1, g2, g3,
                     s0, s1, s2, s3, sem_c):
    bufs = (b0, b1, b2, b3)
    sems_g = (g0, g1, g2, g3)
    sems_s = (s0, s1, s2, s3)
    c = lax.axis_index("c")
    s = lax.axis_index("s")

    pltpu.sync_copy(src_hbm.at[pl.ds(s * NBLK, NBLK)], src_i)
    pltpu.sync_copy(dst_hbm.at[pl.ds(s * NBLK, NBLK)], dst_i)

    _zero_vmem(zbuf, ZR, H)
    for t in range(RPS // ZR):
        pltpu.sync_copy(zbuf, agg_sh.at[pl.ds(s * RPS + t * ZR, ZR)])

    @pl.when(c == 0)
    def _():
        _zero_vmem(zcnt, ZR, 16)

        def orow(i, carry):
            ones_v[i, :] = jnp.ones((16,), f32)
            return carry
        lax.fori_loop(0, B, orow, 0)
        for t in range(RPS // ZR):
            pltpu.sync_copy(zcnt, cnt_sh.at[pl.ds(s * RPS + t * ZR, ZR)])

    plsc.subcore_barrier()

    @pl.when(c == 0)
    def _():
        _edge_loop(ua_hbm, src_i, dst_i, bufs, agg_sh, sems_g, sems_s, 2,
                   cnt_sh, ones_v, sem_c)

    @pl.when(c == 1)
    def _():
        _edge_loop(ub_hbm, src_i, dst_i, bufs, agg_sh, sems_g, sems_s, 2)

    plsc.subcore_barrier()

    pltpu.sync_copy(agg_sh.at[pl.ds(s * RPS, RPS)],
                    agg_out.at[c, pl.ds(s * RPS, RPS)])

    @pl.when(c == 0)
    def _():
        pltpu.sync_copy(cnt_sh.at[pl.ds(s * RPS, RPS)],
                        cnt_out.at[pl.ds(s * RPS, RPS)])


def _sc_agg_body(ua_hbm, ub_hbm, src_hbm, dst_hbm, agg_out,
                 src_i, dst_i, b0, b1, b2, b3, b4,
                 zbuf, agg_sh,
                 g0, g1, g2, g3, g4,
                 s0, s1, s2, s3, s4):
    bufs = (b0, b1, b2, b3, b4)
    sems_g = (g0, g1, g2, g3, g4)
    sems_s = (s0, s1, s2, s3, s4)
    c = lax.axis_index("c")
    s = lax.axis_index("s")

    pltpu.sync_copy(src_hbm.at[pl.ds(s * NBLK, NBLK)], src_i)
    pltpu.sync_copy(dst_hbm.at[pl.ds(s * NBLK, NBLK)], dst_i)

    _zero_vmem(zbuf, ZR, H)
    for t in range(RPS // ZR):
        pltpu.sync_copy(zbuf, agg_sh.at[pl.ds(s * RPS + t * ZR, ZR)])

    plsc.subcore_barrier()

    @pl.when(c == 0)
    def _():
        _edge_loop(ua_hbm, src_i, dst_i, bufs, agg_sh, sems_g, sems_s, 3)

    @pl.when(c == 1)
    def _():
        _edge_loop(ub_hbm, src_i, dst_i, bufs, agg_sh, sems_g, sems_s, 3)

    plsc.subcore_barrier()

    pltpu.sync_copy(agg_sh.at[pl.ds(s * RPS, RPS)],
                    agg_out.at[c, pl.ds(s * RPS, RPS)])


_sc_agg_cnt = pl.kernel(
    _sc_agg_cnt_body,
    out_type=(jax.ShapeDtypeStruct((NC, NP, H), f32),
              jax.ShapeDtypeStruct((NP, 16), f32)),
    mesh=_mesh,
    compiler_params=pltpu.CompilerParams(use_tc_tiling_on_sc=False),
    scratch_types=[
        pltpu.VMEM((NBLK, B), jnp.int32),
        pltpu.VMEM((NBLK, B), jnp.int32),
        pltpu.VMEM((B, H), f32),
        pltpu.VMEM((B, H), f32),
        pltpu.VMEM((B, H), f32),
        pltpu.VMEM((B, H), f32),
        pltpu.VMEM((ZR, H), f32),
        pltpu.VMEM((ZR, 16), f32),
        pltpu.VMEM((B, 16), f32),
        pltpu.VMEM_SHARED((NP, H), f32),
        pltpu.VMEM_SHARED((NP, 16), f32),
        pltpu.SemaphoreType.DMA,
        pltpu.SemaphoreType.DMA,
        pltpu.SemaphoreType.DMA,
        pltpu.SemaphoreType.DMA,
        pltpu.SemaphoreType.DMA,
        pltpu.SemaphoreType.DMA,
        pltpu.SemaphoreType.DMA,
        pltpu.SemaphoreType.DMA,
        pltpu.SemaphoreType.DMA,
    ],
)

_sc_agg = pl.kernel(
    _sc_agg_body,
    out_type=jax.ShapeDtypeStruct((NC, NP, H), f32),
    mesh=_mesh,
    compiler_params=pltpu.CompilerParams(use_tc_tiling_on_sc=False),
    scratch_types=[
        pltpu.VMEM((NBLK, B), jnp.int32),
        pltpu.VMEM((NBLK, B), jnp.int32),
        pltpu.VMEM((B, H), f32),
        pltpu.VMEM((B, H), f32),
        pltpu.VMEM((B, H), f32),
        pltpu.VMEM((B, H), f32),
        pltpu.VMEM((B, H), f32),
        pltpu.VMEM((ZR, H), f32),
        pltpu.VMEM_SHARED((NP, H), f32),
        pltpu.SemaphoreType.DMA,
        pltpu.SemaphoreType.DMA,
        pltpu.SemaphoreType.DMA,
        pltpu.SemaphoreType.DMA,
        pltpu.SemaphoreType.DMA,
        pltpu.SemaphoreType.DMA,
        pltpu.SemaphoreType.DMA,
        pltpu.SemaphoreType.DMA,
        pltpu.SemaphoreType.DMA,
        pltpu.SemaphoreType.DMA,
    ],
)


R = 1000
G = N // R

_DOT = (((1,), (1,)), ((), ()))


def _write_u_halves(u, ua_ref, ub_ref):
    ua_ref[...] = u[:, :H]
    ub_ref[...] = u[:, H:]


def _tc_first_body(x_ref, wl_ref, wr_ref, bl_ref, ua_ref, ub_ref, v_ref):
    h = x_ref[...]
    u = lax.dot_general(h, wl_ref[...], _DOT, preferred_element_type=f32)
    _write_u_halves(u, ua_ref, ub_ref)
    v_ref[...] = lax.dot_general(h, wr_ref[...], _DOT,
                                 preferred_element_type=f32) + bl_ref[...]


def _tc_mid_body(agg_ref, cnt_ref, vp_ref, wl_ref, wr_ref, bl_ref,
                 ua_ref, ub_ref, v_ref):
    inv = 1.0 / jnp.maximum(cnt_ref[:, 0:1], 1.0)
    mean = jnp.concatenate([agg_ref[0], agg_ref[1]], axis=1) * inv
    h = jnp.maximum(mean + vp_ref[...], 0.0)
    u = lax.dot_general(h, wl_ref[...], _DOT, preferred_element_type=f32)
    _write_u_halves(u, ua_ref, ub_ref)
    v_ref[...] = lax.dot_general(h, wr_ref[...], _DOT,
                                 preferred_element_type=f32) + bl_ref[...]


def _tc_last_body(agg_ref, cnt_ref, vp_ref, out_ref):
    inv = 1.0 / jnp.maximum(cnt_ref[:, 0:1], 1.0)
    mean = jnp.concatenate([agg_ref[0], agg_ref[1]], axis=1) * inv
    out_ref[...] = mean + vp_ref[...]


_row_spec = pl.BlockSpec((R, D), lambda i: (i, 0))
_half_spec = pl.BlockSpec((R, H), lambda i: (i, 0))
_w_spec = pl.BlockSpec((D, D), lambda i: (0, 0))
_b_spec = pl.BlockSpec((1, D), lambda i: (0, 0))
_agg_spec = pl.BlockSpec((NC, R, H), lambda i: (0, i, 0))
_cnt_spec = pl.BlockSpec((R, 16), lambda i: (i, 0))

_u_shapes = [jax.ShapeDtypeStruct((N, H), f32),
             jax.ShapeDtypeStruct((N, H), f32),
             jax.ShapeDtypeStruct((N, D), f32)]

_tc_first = pl.pallas_call(
    _tc_first_body,
    grid=(G,),
    in_specs=[_row_spec, _w_spec, _w_spec, _b_spec],
    out_specs=[_half_spec, _half_spec, _row_spec],
    out_shape=_u_shapes,
)

_tc_mid = pl.pallas_call(
    _tc_mid_body,
    grid=(G,),
    in_specs=[_agg_spec, _cnt_spec, _row_spec, _w_spec, _w_spec, _b_spec],
    out_specs=[_half_spec, _half_spec, _row_spec],
    out_shape=_u_shapes,
)

_tc_last = pl.pallas_call(
    _tc_last_body,
    grid=(G,),
    in_specs=[_agg_spec, _cnt_spec, _row_spec],
    out_specs=_row_spec,
    out_shape=jax.ShapeDtypeStruct((N, D), f32),
)


@jax.jit
def kernel(x, edge_index, Wl1, bl1, Wr1, Wl2, bl2, Wr2, Wl3, bl3, Wr3):
    pad_src = jnp.zeros((EP - E,), jnp.int32)
    pad_dst = jnp.full((EP - E,), N, jnp.int32)
    src = jnp.concatenate([edge_index[0], pad_src]).reshape(EP // B, B)
    dst = jnp.concatenate([edge_index[1], pad_dst]).reshape(EP // B, B)

    ua1, ub1, v1 = _tc_first(x, Wl1, Wr1, bl1.reshape(1, D))
    agg1, cnt = _sc_agg_cnt(ua1, ub1, src, dst)
    ua2, ub2, v2 = _tc_mid(agg1, cnt, v1, Wl2, Wr2, bl2.reshape(1, D))
    agg2 = _sc_agg(ua2, ub2, src, dst)
    ua3, ub3, v3 = _tc_mid(agg2, cnt, v2, Wl3, Wr3, bl3.reshape(1, D))
    agg3 = _sc_agg(ua3, ub3, src, dst)
    return _tc_last(agg3, cnt, v3)


# speedup vs baseline: 1.0108x; 1.0108x over previous
import jax
import jax.numpy as jnp
from jax import lax
from jax.experimental import pallas as pl
from jax.experimental.pallas import tpu as pltpu
from jax.experimental.pallas import tpu_sc as plsc

N = 10000
E = 320000
D = 128
H = D // 2

NC = 2
NS = 16
B = 128
EP = 327680
EPT = EP // NS
NBLK = EPT // B
NP = 10240
RPS = NP // NS
ZR = 32

f32 = jnp.float32

_mesh = plsc.VectorSubcoreMesh(core_axis_name="c", subcore_axis_name="s")


def _zero_vmem(buf, rows, width):
    def zrow(i, carry):
        for j in range(width // 16):
            buf[i, pl.ds(j * 16, 16)] = jnp.zeros((16,), f32)
        return carry
    lax.fori_loop(0, rows, zrow, 0)


def _edge_loop(u_hbm, src_i, dst_i, bufs, agg_sh, sems_g, sems_s, lead,
               cnt_sh=None, ones_v=None, sem_c=None):
    nslot = len(bufs)
    for k in range(lead):
        pltpu.async_copy(u_hbm.at[src_i.at[k]], bufs[k], sems_g[k])

    dwait = nslot - lead

    def grp(g, carry):
        t0 = g * nslot
        for k in range(nslot):
            t = t0 + k
            buf, sem_g, sem_s = bufs[k], sems_g[k], sems_s[k]
            pltpu.make_async_copy(u_hbm.at[src_i.at[t]], buf, sem_g).wait()
            pltpu.async_copy(buf, agg_sh.at[dst_i.at[t]], sem_s, add=True)
            if cnt_sh is not None:
                pltpu.async_copy(ones_v, cnt_sh.at[dst_i.at[t]], sem_c,
                                 add=True)

            kw = (k - dwait) % nslot

            @pl.when(t >= dwait)
            def _():
                pltpu.make_async_copy(bufs[kw], agg_sh.at[dst_i.at[t]],
                                      sems_s[kw]).wait()

            kn = (k + lead) % nslot

            @pl.when(t + lead < NBLK)
            def _():
                pltpu.async_copy(u_hbm.at[src_i.at[t + lead]], bufs[kn],
                                 sems_g[kn])
        return carry
    lax.fori_loop(0, NBLK // nslot, grp, 0)

    for t in range(NBLK - dwait, NBLK):
        k = t % nslot
        pltpu.make_async_copy(bufs[k], agg_sh.at[dst_i.at[0]],
                              sems_s[k]).wait()

    if cnt_sh is not None:
        def drain(i, carry):
            pltpu.make_async_copy(ones_v, cnt_sh.at[dst_i.at[0]],
                                  sem_c).wait()
            return carry
        lax.fori_loop(0, NBLK, drain, 0)


def _sc_agg_cnt_body(ua_hbm, ub_hbm, src_hbm, dst_hbm, agg_out, cnt_out,
                     src_i, dst_i, b0, b1, b2, b3,
                     zbuf, zcnt, ones_v,
                     agg_sh, cnt_sh, g0, g1, g2, g3,
                     s0, s1, s2, s3, sem_c):
    bufs = (b0, b1, b2, b3)
    sems_g = (g0, g1, g2, g3)
    sems_s = (s0, s1, s2, s3)
    c = lax.axis_index("c")
    s = lax.axis_index("s")

    pltpu.sync_copy(src_hbm.at[pl.ds(s * NBLK, NBLK)], src_i)
    pltpu.sync_copy(dst_hbm.at[pl.ds(s * NBLK, NBLK)], dst_i)

    _zero_vmem(zbuf, ZR, H)
    for t in range(RPS // ZR):
        pltpu.sync_copy(zbuf, agg_sh.at[pl.ds(s * RPS + t * ZR, ZR)])

    @pl.when(c == 0)
    def _():
        _zero_vmem(zcnt, ZR, 16)

        def orow(i, carry):
            ones_v[i, :] = jnp.ones((16,), f32)
            return carry
        lax.fori_loop(0, B, orow, 0)
        for t in range(RPS // ZR):
            pltpu.sync_copy(zcnt, cnt_sh.at[pl.ds(s * RPS + t * ZR, ZR)])

    plsc.subcore_barrier()

    @pl.when(c == 0)
    def _():
        _edge_loop(ua_hbm, src_i, dst_i, bufs, agg_sh, sems_g, sems_s, 2,
                   cnt_sh, ones_v, sem_c)

    @pl.when(c == 1)
    def _():
        _edge_loop(ub_hbm, src_i, dst_i, bufs, agg_sh, sems_g, sems_s, 2)

    plsc.subcore_barrier()

    pltpu.sync_copy(agg_sh.at[pl.ds(s * RPS, RPS)],
                    agg_out.at[c, pl.ds(s * RPS, RPS)])

    @pl.when(c == 0)
    def _():
        pltpu.sync_copy(cnt_sh.at[pl.ds(s * RPS, RPS)],
                        cnt_out.at[pl.ds(s * RPS, RPS)])


def _sc_agg_body(ua_hbm, ub_hbm, src_hbm, dst_hbm, agg_out,
                 src_i, dst_i, b0, b1, b2, b3, b4,
                 zbuf, agg_sh,
                 g0, g1, g2, g3, g4,
                 s0, s1, s2, s3, s4):
    bufs = (b0, b1, b2, b3, b4)
    sems_g = (g0, g1, g2, g3, g4)
    sems_s = (s0, s1, s2, s3, s4)
    c = lax.axis_index("c")
    s = lax.axis_index("s")

    pltpu.sync_copy(src_hbm.at[pl.ds(s * NBLK, NBLK)], src_i)
    pltpu.sync_copy(dst_hbm.at[pl.ds(s * NBLK, NBLK)], dst_i)

    _zero_vmem(zbuf, ZR, H)
    for t in range(RPS // ZR):
        pltpu.sync_copy(zbuf, agg_sh.at[pl.ds(s * RPS + t * ZR, ZR)])

    plsc.subcore_barrier()

    @pl.when(c == 0)
    def _():
        _edge_loop(ua_hbm, src_i, dst_i, bufs, agg_sh, sems_g, sems_s, 3)

    @pl.when(c == 1)
    def _():
        _edge_loop(ub_hbm, src_i, dst_i, bufs, agg_sh, sems_g, sems_s, 3)

    plsc.subcore_barrier()

    pltpu.sync_copy(agg_sh.at[pl.ds(s * RPS, RPS)],
                    agg_out.at[c, pl.ds(s * RPS, RPS)])


_sc_agg_cnt = pl.kernel(
    _sc_agg_cnt_body,
    out_type=(jax.ShapeDtypeStruct((NC, NP, H), f32),
              jax.ShapeDtypeStruct((NP, 16), f32)),
    mesh=_mesh,
    compiler_params=pltpu.CompilerParams(use_tc_tiling_on_sc=False),
    scratch_types=[
        pltpu.VMEM((NBLK, B), jnp.int32),
        pltpu.VMEM((NBLK, B), jnp.int32),
        pltpu.VMEM((B, H), f32),
        pltpu.VMEM((B, H), f32),
        pltpu.VMEM((B, H), f32),
        pltpu.VMEM((B, H), f32),
        pltpu.VMEM((ZR, H), f32),
        pltpu.VMEM((ZR, 16), f32),
        pltpu.VMEM((B, 16), f32),
        pltpu.VMEM_SHARED((NP, H), f32),
        pltpu.VMEM_SHARED((NP, 16), f32),
        pltpu.SemaphoreType.DMA,
        pltpu.SemaphoreType.DMA,
        pltpu.SemaphoreType.DMA,
        pltpu.SemaphoreType.DMA,
        pltpu.SemaphoreType.DMA,
        pltpu.SemaphoreType.DMA,
        pltpu.SemaphoreType.DMA,
        pltpu.SemaphoreType.DMA,
        pltpu.SemaphoreType.DMA,
    ],
)

_sc_agg = pl.kernel(
    _sc_agg_body,
    out_type=jax.ShapeDtypeStruct((NC, NP, H), f32),
    mesh=_mesh,
    compiler_params=pltpu.CompilerParams(use_tc_tiling_on_sc=False),
    scratch_types=[
        pltpu.VMEM((NBLK, B), jnp.int32),
        pltpu.VMEM((NBLK, B), jnp.int32),
        pltpu.VMEM((B, H), f32),
        pltpu.VMEM((B, H), f32),
        pltpu.VMEM((B, H), f32),
        pltpu.VMEM((B, H), f32),
        pltpu.VMEM((B, H), f32),
        pltpu.VMEM((ZR, H), f32),
        pltpu.VMEM_SHARED((NP, H), f32),
        pltpu.SemaphoreType.DMA,
        pltpu.SemaphoreType.DMA,
        pltpu.SemaphoreType.DMA,
        pltpu.SemaphoreType.DMA,
        pltpu.SemaphoreType.DMA,
        pltpu.SemaphoreType.DMA,
        pltpu.SemaphoreType.DMA,
        pltpu.SemaphoreType.DMA,
        pltpu.SemaphoreType.DMA,
        pltpu.SemaphoreType.DMA,
    ],
)


R = 1000
G = N // R

_DOT = (((1,), (1,)), ((), ()))


def _write_u_halves(u, ua_ref, ub_ref):
    ua_ref[...] = u[:, :H]
    ub_ref[...] = u[:, H:]


def _tc_first_body(x_ref, wl_ref, wr_ref, bl_ref, ua_ref, ub_ref, v_ref):
    h = x_ref[...]
    u = lax.dot_general(h, wl_ref[...], _DOT, preferred_element_type=f32)
    _write_u_halves(u, ua_ref, ub_ref)
    v_ref[...] = lax.dot_general(h, wr_ref[...], _DOT,
                                 preferred_element_type=f32) + bl_ref[...]


def _tc_mid_body(agg_ref, cnt_ref, vp_ref, wl_ref, wr_ref, bl_ref,
                 ua_ref, ub_ref, v_ref):
    inv = 1.0 / jnp.maximum(cnt_ref[:, 0:1], 1.0)
    mean = jnp.concatenate([agg_ref[0], agg_ref[1]], axis=1) * inv
    h = jnp.maximum(mean + vp_ref[...], 0.0)
    u = lax.dot_general(h, wl_ref[...], _DOT, preferred_element_type=f32)
    _write_u_halves(u, ua_ref, ub_ref)
    v_ref[...] = lax.dot_general(h, wr_ref[...], _DOT,
                                 preferred_element_type=f32) + bl_ref[...]


def _tc_last_body(agg_ref, cnt_ref, vp_ref, out_ref):
    inv = 1.0 / jnp.maximum(cnt_ref[:, 0:1], 1.0)
    mean = jnp.concatenate([agg_ref[0], agg_ref[1]], axis=1) * inv
    out_ref[...] = mean + vp_ref[...]


_row_spec = pl.BlockSpec((R, D), lambda i: (i, 0))
_half_spec = pl.BlockSpec((R, H), lambda i: (i, 0))
_w_spec = pl.BlockSpec((D, D), lambda i: (0, 0))
_b_spec = pl.BlockSpec((1, D), lambda i: (0, 0))
_agg_spec = pl.BlockSpec((NC, R, H), lambda i: (0, i, 0))
_cnt_spec = pl.BlockSpec((R, 16), lambda i: (i, 0))

_u_shapes = [jax.ShapeDtypeStruct((N, H), f32),
             jax.ShapeDtypeStruct((N, H), f32),
             jax.ShapeDtypeStruct((N, D), f32)]

_tc_first = pl.pallas_call(
    _tc_first_body,
    grid=(G,),
    in_specs=[_row_spec, _w_spec, _w_spec, _b_spec],
    out_specs=[_half_spec, _half_spec, _row_spec],
    out_shape=_u_shapes,
)

_tc_mid = pl.pallas_call(
    _tc_mid_body,
    grid=(G,),
    in_specs=[_agg_spec, _cnt_spec, _row_spec, _w_spec, _w_spec, _b_spec],
    out_specs=[_half_spec, _half_spec, _row_spec],
    out_shape=_u_shapes,
)

_tc_last = pl.pallas_call(
    _tc_last_body,
    grid=(G,),
    in_specs=[_agg_spec, _cnt_spec, _row_spec],
    out_specs=_row_spec,
    out_shape=jax.ShapeDtypeStruct((N, D), f32),
)


@jax.jit
def kernel(x, edge_index, Wl1, bl1, Wr1, Wl2, bl2, Wr2, Wl3, bl3, Wr3):
    pad_src = jnp.zeros((EP - E,), jnp.int32)
    pad_dst = N + jax.lax.rem(jnp.arange(EP - E, dtype=jnp.int32),
                              jnp.int32(NP - N))
    src = jnp.concatenate([edge_index[0], pad_src]).reshape(EP // B, B)
    dst = jnp.concatenate([edge_index[1], pad_dst]).reshape(EP // B, B)

    ua1, ub1, v1 = _tc_first(x, Wl1, Wr1, bl1.reshape(1, D))
    agg1, cnt = _sc_agg_cnt(ua1, ub1, src, dst)
    ua2, ub2, v2 = _tc_mid(agg1, cnt, v1, Wl2, Wr2, bl2.reshape(1, D))
    agg2 = _sc_agg(ua2, ub2, src, dst)
    ua3, ub3, v3 = _tc_mid(agg2, cnt, v2, Wl3, Wr3, bl3.reshape(1, D))
    agg3 = _sc_agg(ua3, ub3, src, dst)
    return _tc_last(agg3, cnt, v3)


# speedup vs baseline: 2.3713x; 2.3459x over previous
import jax
import jax.numpy as jnp
from jax import lax
from jax.experimental import pallas as pl
from jax.experimental.pallas import tpu as pltpu
from jax.experimental.pallas import tpu_sc as plsc

N = 10000
E = 320000
D = 128
H = D // 2

NC = 2
NS = 16
B = 80
EP = E
EPT = EP // NS
NBLK = EPT // B
NP = 10240
RPS = NP // NS
ZR = 32

f32 = jnp.float32

_mesh = plsc.VectorSubcoreMesh(core_axis_name="c", subcore_axis_name="s")


def _zero_vmem(buf, rows, width):
    def zrow(i, carry):
        for j in range(width // 16):
            buf[i, pl.ds(j * 16, 16)] = jnp.zeros((16,), f32)
        return carry
    lax.fori_loop(0, rows, zrow, 0)


def _edge_loop(u_hbm, src_i, dst_i, bufs, agg_sh, sems_g, sems_s, lead,
               cnt_sh=None, ones_v=None, sem_c=None):
    nslot = len(bufs)
    for k in range(lead):
        pltpu.async_copy(u_hbm.at[src_i.at[k]], bufs[k], sems_g[k])

    dwait = nslot - lead

    def grp(g, carry):
        t0 = g * nslot
        for k in range(nslot):
            t = t0 + k
            buf, sem_g, sem_s = bufs[k], sems_g[k], sems_s[k]
            pltpu.make_async_copy(u_hbm.at[src_i.at[t]], buf, sem_g).wait()
            pltpu.async_copy(buf, agg_sh.at[dst_i.at[t]], sem_s, add=True)
            if cnt_sh is not None:
                pltpu.async_copy(ones_v, cnt_sh.at[dst_i.at[t]], sem_c,
                                 add=True)

            kw = (k - dwait) % nslot

            @pl.when(t >= dwait)
            def _():
                pltpu.make_async_copy(bufs[kw], agg_sh.at[dst_i.at[t]],
                                      sems_s[kw]).wait()

            kn = (k + lead) % nslot

            @pl.when(t + lead < NBLK)
            def _():
                pltpu.async_copy(u_hbm.at[src_i.at[t + lead]], bufs[kn],
                                 sems_g[kn])
        return carry
    lax.fori_loop(0, NBLK // nslot, grp, 0)

    for t in range(NBLK - dwait, NBLK):
        k = t % nslot
        pltpu.make_async_copy(bufs[k], agg_sh.at[dst_i.at[0]],
                              sems_s[k]).wait()

    if cnt_sh is not None:
        def drain(i, carry):
            pltpu.make_async_copy(ones_v, cnt_sh.at[dst_i.at[0]],
                                  sem_c).wait()
            return carry
        lax.fori_loop(0, NBLK, drain, 0)


def _sc_agg_cnt_body(ua_hbm, ub_hbm, src_hbm, dst_hbm, agg_out, cnt_out,
                     src_i, dst_i, b0, b1, b2, b3, b4,
                     zbuf, zcnt, ones_v,
                     agg_sh, cnt_sh, g0, g1, g2, g3, g4,
                     s0, s1, s2, s3, s4, sem_c):
    bufs = (b0, b1, b2, b3, b4)
    sems_g = (g0, g1, g2, g3, g4)
    sems_s = (s0, s1, s2, s3, s4)
    c = lax.axis_index("c")
    s = lax.axis_index("s")

    pltpu.sync_copy(src_hbm.at[pl.ds(s * NBLK, NBLK)], src_i)
    pltpu.sync_copy(dst_hbm.at[pl.ds(s * NBLK, NBLK)], dst_i)

    _zero_vmem(zbuf, ZR, H)
    for t in range(RPS // ZR):
        pltpu.sync_copy(zbuf, agg_sh.at[pl.ds(s * RPS + t * ZR, ZR)])

    @pl.when(c == 0)
    def _():
        _zero_vmem(zcnt, ZR, 16)

        def orow(i, carry):
            ones_v[i, :] = jnp.ones((16,), f32)
            return carry
        lax.fori_loop(0, B, orow, 0)
        for t in range(RPS // ZR):
            pltpu.sync_copy(zcnt, cnt_sh.at[pl.ds(s * RPS + t * ZR, ZR)])

    plsc.subcore_barrier()

    @pl.when(c == 0)
    def _():
        _edge_loop(ua_hbm, src_i, dst_i, bufs, agg_sh, sems_g, sems_s, 3,
                   cnt_sh, ones_v, sem_c)

    @pl.when(c == 1)
    def _():
        _edge_loop(ub_hbm, src_i, dst_i, bufs, agg_sh, sems_g, sems_s, 3)

    plsc.subcore_barrier()

    pltpu.sync_copy(agg_sh.at[pl.ds(s * RPS, RPS)],
                    agg_out.at[c, pl.ds(s * RPS, RPS)])

    @pl.when(c == 0)
    def _():
        pltpu.sync_copy(cnt_sh.at[pl.ds(s * RPS, RPS)],
                        cnt_out.at[pl.ds(s * RPS, RPS)])


def _sc_agg_body(ua_hbm, ub_hbm, src_hbm, dst_hbm, agg_out,
                 src_i, dst_i, b0, b1, b2, b3, b4,
                 zbuf, agg_sh,
                 g0, g1, g2, g3, g4,
                 s0, s1, s2, s3, s4):
    bufs = (b0, b1, b2, b3, b4)
    sems_g = (g0, g1, g2, g3, g4)
    sems_s = (s0, s1, s2, s3, s4)
    c = lax.axis_index("c")
    s = lax.axis_index("s")

    pltpu.sync_copy(src_hbm.at[pl.ds(s * NBLK, NBLK)], src_i)
    pltpu.sync_copy(dst_hbm.at[pl.ds(s * NBLK, NBLK)], dst_i)

    _zero_vmem(zbuf, ZR, H)
    for t in range(RPS // ZR):
        pltpu.sync_copy(zbuf, agg_sh.at[pl.ds(s * RPS + t * ZR, ZR)])

    plsc.subcore_barrier()

    @pl.when(c == 0)
    def _():
        _edge_loop(ua_hbm, src_i, dst_i, bufs, agg_sh, sems_g, sems_s, 3)

    @pl.when(c == 1)
    def _():
        _edge_loop(ub_hbm, src_i, dst_i, bufs, agg_sh, sems_g, sems_s, 3)

    plsc.subcore_barrier()

    pltpu.sync_copy(agg_sh.at[pl.ds(s * RPS, RPS)],
                    agg_out.at[c, pl.ds(s * RPS, RPS)])


_sc_agg_cnt = pl.kernel(
    _sc_agg_cnt_body,
    out_type=(jax.ShapeDtypeStruct((NC, NP, H), f32),
              jax.ShapeDtypeStruct((NP, 16), f32)),
    mesh=_mesh,
    compiler_params=pltpu.CompilerParams(use_tc_tiling_on_sc=False),
    scratch_types=[
        pltpu.VMEM((NBLK, B), jnp.int32),
        pltpu.VMEM((NBLK, B), jnp.int32),
        pltpu.VMEM((B, H), f32),
        pltpu.VMEM((B, H), f32),
        pltpu.VMEM((B, H), f32),
        pltpu.VMEM((B, H), f32),
        pltpu.VMEM((B, H), f32),
        pltpu.VMEM((ZR, H), f32),
        pltpu.VMEM((ZR, 16), f32),
        pltpu.VMEM((B, 16), f32),
        pltpu.VMEM_SHARED((NP, H), f32),
        pltpu.VMEM_SHARED((NP, 16), f32),
        pltpu.SemaphoreType.DMA,
        pltpu.SemaphoreType.DMA,
        pltpu.SemaphoreType.DMA,
        pltpu.SemaphoreType.DMA,
        pltpu.SemaphoreType.DMA,
        pltpu.SemaphoreType.DMA,
        pltpu.SemaphoreType.DMA,
        pltpu.SemaphoreType.DMA,
        pltpu.SemaphoreType.DMA,
        pltpu.SemaphoreType.DMA,
        pltpu.SemaphoreType.DMA,
    ],
)

_sc_agg = pl.kernel(
    _sc_agg_body,
    out_type=jax.ShapeDtypeStruct((NC, NP, H), f32),
    mesh=_mesh,
    compiler_params=pltpu.CompilerParams(use_tc_tiling_on_sc=False),
    scratch_types=[
        pltpu.VMEM((NBLK, B), jnp.int32),
        pltpu.VMEM((NBLK, B), jnp.int32),
        pltpu.VMEM((B, H), f32),
        pltpu.VMEM((B, H), f32),
        pltpu.VMEM((B, H), f32),
        pltpu.VMEM((B, H), f32),
        pltpu.VMEM((B, H), f32),
        pltpu.VMEM((ZR, H), f32),
        pltpu.VMEM_SHARED((NP, H), f32),
        pltpu.SemaphoreType.DMA,
        pltpu.SemaphoreType.DMA,
        pltpu.SemaphoreType.DMA,
        pltpu.SemaphoreType.DMA,
        pltpu.SemaphoreType.DMA,
        pltpu.SemaphoreType.DMA,
        pltpu.SemaphoreType.DMA,
        pltpu.SemaphoreType.DMA,
        pltpu.SemaphoreType.DMA,
        pltpu.SemaphoreType.DMA,
    ],
)


R = 1000
G = N // R

_DOT = (((1,), (1,)), ((), ()))


def _write_u_halves(u, ua_ref, ub_ref):
    ua_ref[...] = u[:, :H]
    ub_ref[...] = u[:, H:]


def _tc_first_body(x_ref, wl_ref, wr_ref, bl_ref, ua_ref, ub_ref, v_ref):
    h = x_ref[...]
    u = lax.dot_general(h, wl_ref[...], _DOT, preferred_element_type=f32)
    _write_u_halves(u, ua_ref, ub_ref)
    v_ref[...] = lax.dot_general(h, wr_ref[...], _DOT,
                                 preferred_element_type=f32) + bl_ref[...]


def _tc_mid_body(agg_ref, cnt_ref, vp_ref, wl_ref, wr_ref, bl_ref,
                 ua_ref, ub_ref, v_ref):
    inv = 1.0 / jnp.maximum(cnt_ref[:, 0:1], 1.0)
    mean = jnp.concatenate([agg_ref[0], agg_ref[1]], axis=1) * inv
    h = jnp.maximum(mean + vp_ref[...], 0.0)
    u = lax.dot_general(h, wl_ref[...], _DOT, preferred_element_type=f32)
    _write_u_halves(u, ua_ref, ub_ref)
    v_ref[...] = lax.dot_general(h, wr_ref[...], _DOT,
                                 preferred_element_type=f32) + bl_ref[...]


def _tc_last_body(agg_ref, cnt_ref, vp_ref, out_ref):
    inv = 1.0 / jnp.maximum(cnt_ref[:, 0:1], 1.0)
    mean = jnp.concatenate([agg_ref[0], agg_ref[1]], axis=1) * inv
    out_ref[...] = mean + vp_ref[...]


_row_spec = pl.BlockSpec((R, D), lambda i: (i, 0))
_half_spec = pl.BlockSpec((R, H), lambda i: (i, 0))
_w_spec = pl.BlockSpec((D, D), lambda i: (0, 0))
_b_spec = pl.BlockSpec((1, D), lambda i: (0, 0))
_agg_spec = pl.BlockSpec((NC, R, H), lambda i: (0, i, 0))
_cnt_spec = pl.BlockSpec((R, 16), lambda i: (i, 0))

_u_shapes = [jax.ShapeDtypeStruct((N, H), f32),
             jax.ShapeDtypeStruct((N, H), f32),
             jax.ShapeDtypeStruct((N, D), f32)]

_tc_first = pl.pallas_call(
    _tc_first_body,
    grid=(G,),
    in_specs=[_row_spec, _w_spec, _w_spec, _b_spec],
    out_specs=[_half_spec, _half_spec, _row_spec],
    out_shape=_u_shapes,
)

_tc_mid = pl.pallas_call(
    _tc_mid_body,
    grid=(G,),
    in_specs=[_agg_spec, _cnt_spec, _row_spec, _w_spec, _w_spec, _b_spec],
    out_specs=[_half_spec, _half_spec, _row_spec],
    out_shape=_u_shapes,
)

_tc_last = pl.pallas_call(
    _tc_last_body,
    grid=(G,),
    in_specs=[_agg_spec, _cnt_spec, _row_spec],
    out_specs=_row_spec,
    out_shape=jax.ShapeDtypeStruct((N, D), f32),
)


@jax.jit
def kernel(x, edge_index, Wl1, bl1, Wr1, Wl2, bl2, Wr2, Wl3, bl3, Wr3):
    src = edge_index[0].reshape(EP // B, B)
    dst = edge_index[1].reshape(EP // B, B)

    ua1, ub1, v1 = _tc_first(x, Wl1, Wr1, bl1.reshape(1, D))
    agg1, cnt = _sc_agg_cnt(ua1, ub1, src, dst)
    ua2, ub2, v2 = _tc_mid(agg1, cnt, v1, Wl2, Wr2, bl2.reshape(1, D))
    agg2 = _sc_agg(ua2, ub2, src, dst)
    ua3, ub3, v3 = _tc_mid(agg2, cnt, v2, Wl3, Wr3, bl3.reshape(1, D))
    agg3 = _sc_agg(ua3, ub3, src, dst)
    return _tc_last(agg3, cnt, v3)


# speedup vs baseline: 2.5016x; 1.0549x over previous
import jax
import jax.numpy as jnp
from jax import lax
from jax.experimental import pallas as pl
from jax.experimental.pallas import tpu as pltpu
from jax.experimental.pallas import tpu_sc as plsc

N = 10000
E = 320000
D = 128
H = D // 2

NC = 2
NS = 16
B = 80
EP = E
EPT = EP // NS
NBLK = EPT // B
NP = 10240
RPS = NP // NS
ZR = 32

f32 = jnp.float32

_mesh = plsc.VectorSubcoreMesh(core_axis_name="c", subcore_axis_name="s")


def _zero_vmem(buf, rows, width):
    def zrow(i, carry):
        for j in range(width // 16):
            buf[i, pl.ds(j * 16, 16)] = jnp.zeros((16,), f32)
        return carry
    lax.fori_loop(0, rows, zrow, 0)


def _edge_loop(u_hbm, src_i, dst_i, bufs, agg_sh, sems_g, sems_s, lead,
               cnt_sh=None, ones_v=None, sem_c=None):
    nslot = len(bufs)
    for k in range(lead):
        pltpu.async_copy(u_hbm.at[src_i.at[k]], bufs[k], sems_g[k])

    dwait = nslot - lead

    def grp(g, carry):
        t0 = g * nslot
        for k in range(nslot):
            t = t0 + k
            buf, sem_g, sem_s = bufs[k], sems_g[k], sems_s[k]
            pltpu.make_async_copy(u_hbm.at[src_i.at[t]], buf, sem_g).wait()
            pltpu.async_copy(buf, agg_sh.at[dst_i.at[t]], sem_s, add=True)
            if cnt_sh is not None:
                pltpu.async_copy(ones_v, cnt_sh.at[dst_i.at[t]], sem_c,
                                 add=True)

            kw = (k - dwait) % nslot

            @pl.when(t >= dwait)
            def _():
                pltpu.make_async_copy(bufs[kw], agg_sh.at[dst_i.at[t]],
                                      sems_s[kw]).wait()

            kn = (k + lead) % nslot

            @pl.when(t + lead < NBLK)
            def _():
                pltpu.async_copy(u_hbm.at[src_i.at[t + lead]], bufs[kn],
                                 sems_g[kn])
        return carry
    lax.fori_loop(0, NBLK // nslot, grp, 0)

    for t in range(NBLK - dwait, NBLK):
        k = t % nslot
        pltpu.make_async_copy(bufs[k], agg_sh.at[dst_i.at[0]],
                              sems_s[k]).wait()

    if cnt_sh is not None:
        def drain(i, carry):
            pltpu.make_async_copy(ones_v, cnt_sh.at[dst_i.at[0]],
                                  sem_c).wait()
            return carry
        lax.fori_loop(0, NBLK, drain, 0)


def _sc_agg_cnt_body(ua_hbm, ub_hbm, src_hbm, dst_hbm, agg_out, cnt_out,
                     src_i, dst_i, b0, b1, b2, b3, b4,
                     zbuf, zcnt, ones_v,
                     agg_sh, cnt_sh, g0, g1, g2, g3, g4,
                     s0, s1, s2, s3, s4, sem_c):
    bufs = (b0, b1, b2, b3, b4)
    sems_g = (g0, g1, g2, g3, g4)
    sems_s = (s0, s1, s2, s3, s4)
    c = lax.axis_index("c")
    s = lax.axis_index("s")

    pltpu.sync_copy(src_hbm.at[pl.ds(s * NBLK, NBLK)], src_i)
    pltpu.sync_copy(dst_hbm.at[pl.ds(s * NBLK, NBLK)], dst_i)

    _zero_vmem(zbuf, ZR, H)
    for t in range(RPS // ZR):
        pltpu.sync_copy(zbuf, agg_sh.at[pl.ds(s * RPS + t * ZR, ZR)])

    @pl.when(c == 0)
    def _():
        _zero_vmem(zcnt, ZR, 16)

        def orow(i, carry):
            ones_v[i, :] = jnp.ones((16,), f32)
            return carry
        lax.fori_loop(0, B, orow, 0)
        for t in range(RPS // ZR):
            pltpu.sync_copy(zcnt, cnt_sh.at[pl.ds(s * RPS + t * ZR, ZR)])

    plsc.subcore_barrier()

    @pl.when(c == 0)
    def _():
        _edge_loop(ua_hbm, src_i, dst_i, bufs, agg_sh, sems_g, sems_s, 4,
                   cnt_sh, ones_v, sem_c)

    @pl.when(c == 1)
    def _():
        _edge_loop(ub_hbm, src_i, dst_i, bufs, agg_sh, sems_g, sems_s, 4)

    plsc.subcore_barrier()

    pltpu.sync_copy(agg_sh.at[pl.ds(s * RPS, RPS)],
                    agg_out.at[c, pl.ds(s * RPS, RPS)])

    @pl.when(c == 0)
    def _():
        pltpu.sync_copy(cnt_sh.at[pl.ds(s * RPS, RPS)],
                        cnt_out.at[pl.ds(s * RPS, RPS)])


def _sc_agg_body(ua_hbm, ub_hbm, src_hbm, dst_hbm, agg_out,
                 src_i, dst_i, b0, b1, b2, b3, b4,
                 zbuf, agg_sh,
                 g0, g1, g2, g3, g4,
                 s0, s1, s2, s3, s4):
    bufs = (b0, b1, b2, b3, b4)
    sems_g = (g0, g1, g2, g3, g4)
    sems_s = (s0, s1, s2, s3, s4)
    c = lax.axis_index("c")
    s = lax.axis_index("s")

    pltpu.sync_copy(src_hbm.at[pl.ds(s * NBLK, NBLK)], src_i)
    pltpu.sync_copy(dst_hbm.at[pl.ds(s * NBLK, NBLK)], dst_i)

    _zero_vmem(zbuf, ZR, H)
    for t in range(RPS // ZR):
        pltpu.sync_copy(zbuf, agg_sh.at[pl.ds(s * RPS + t * ZR, ZR)])

    plsc.subcore_barrier()

    @pl.when(c == 0)
    def _():
        _edge_loop(ua_hbm, src_i, dst_i, bufs, agg_sh, sems_g, sems_s, 4)

    @pl.when(c == 1)
    def _():
        _edge_loop(ub_hbm, src_i, dst_i, bufs, agg_sh, sems_g, sems_s, 4)

    plsc.subcore_barrier()

    pltpu.sync_copy(agg_sh.at[pl.ds(s * RPS, RPS)],
                    agg_out.at[c, pl.ds(s * RPS, RPS)])


_sc_agg_cnt = pl.kernel(
    _sc_agg_cnt_body,
    out_type=(jax.ShapeDtypeStruct((NC, NP, H), f32),
              jax.ShapeDtypeStruct((NP, 16), f32)),
    mesh=_mesh,
    compiler_params=pltpu.CompilerParams(use_tc_tiling_on_sc=False),
    scratch_types=[
        pltpu.VMEM((NBLK, B), jnp.int32),
        pltpu.VMEM((NBLK, B), jnp.int32),
        pltpu.VMEM((B, H), f32),
        pltpu.VMEM((B, H), f32),
        pltpu.VMEM((B, H), f32),
        pltpu.VMEM((B, H), f32),
        pltpu.VMEM((B, H), f32),
        pltpu.VMEM((ZR, H), f32),
        pltpu.VMEM((ZR, 16), f32),
        pltpu.VMEM((B, 16), f32),
        pltpu.VMEM_SHARED((NP, H), f32),
        pltpu.VMEM_SHARED((NP, 16), f32),
        pltpu.SemaphoreType.DMA,
        pltpu.SemaphoreType.DMA,
        pltpu.SemaphoreType.DMA,
        pltpu.SemaphoreType.DMA,
        pltpu.SemaphoreType.DMA,
        pltpu.SemaphoreType.DMA,
        pltpu.SemaphoreType.DMA,
        pltpu.SemaphoreType.DMA,
        pltpu.SemaphoreType.DMA,
        pltpu.SemaphoreType.DMA,
        pltpu.SemaphoreType.DMA,
    ],
)

_sc_agg = pl.kernel(
    _sc_agg_body,
    out_type=jax.ShapeDtypeStruct((NC, NP, H), f32),
    mesh=_mesh,
    compiler_params=pltpu.CompilerParams(use_tc_tiling_on_sc=False),
    scratch_types=[
        pltpu.VMEM((NBLK, B), jnp.int32),
        pltpu.VMEM((NBLK, B), jnp.int32),
        pltpu.VMEM((B, H), f32),
        pltpu.VMEM((B, H), f32),
        pltpu.VMEM((B, H), f32),
        pltpu.VMEM((B, H), f32),
        pltpu.VMEM((B, H), f32),
        pltpu.VMEM((ZR, H), f32),
        pltpu.VMEM_SHARED((NP, H), f32),
        pltpu.SemaphoreType.DMA,
        pltpu.SemaphoreType.DMA,
        pltpu.SemaphoreType.DMA,
        pltpu.SemaphoreType.DMA,
        pltpu.SemaphoreType.DMA,
        pltpu.SemaphoreType.DMA,
        pltpu.SemaphoreType.DMA,
        pltpu.SemaphoreType.DMA,
        pltpu.SemaphoreType.DMA,
        pltpu.SemaphoreType.DMA,
    ],
)


R = 1000
G = N // R

_DOT = (((1,), (1,)), ((), ()))


def _write_u_halves(u, ua_ref, ub_ref):
    ua_ref[...] = u[:, :H]
    ub_ref[...] = u[:, H:]


def _tc_first_body(x_ref, wl_ref, wr_ref, bl_ref, ua_ref, ub_ref, v_ref):
    h = x_ref[...]
    u = lax.dot_general(h, wl_ref[...], _DOT, preferred_element_type=f32)
    _write_u_halves(u, ua_ref, ub_ref)
    v_ref[...] = lax.dot_general(h, wr_ref[...], _DOT,
                                 preferred_element_type=f32) + bl_ref[...]


def _tc_mid_body(agg_ref, cnt_ref, vp_ref, wl_ref, wr_ref, bl_ref,
                 ua_ref, ub_ref, v_ref):
    inv = 1.0 / jnp.maximum(cnt_ref[:, 0:1], 1.0)
    mean = jnp.concatenate([agg_ref[0], agg_ref[1]], axis=1) * inv
    h = jnp.maximum(mean + vp_ref[...], 0.0)
    u = lax.dot_general(h, wl_ref[...], _DOT, preferred_element_type=f32)
    _write_u_halves(u, ua_ref, ub_ref)
    v_ref[...] = lax.dot_general(h, wr_ref[...], _DOT,
                                 preferred_element_type=f32) + bl_ref[...]


def _tc_last_body(agg_ref, cnt_ref, vp_ref, out_ref):
    inv = 1.0 / jnp.maximum(cnt_ref[:, 0:1], 1.0)
    mean = jnp.concatenate([agg_ref[0], agg_ref[1]], axis=1) * inv
    out_ref[...] = mean + vp_ref[...]


_row_spec = pl.BlockSpec((R, D), lambda i: (i, 0))
_half_spec = pl.BlockSpec((R, H), lambda i: (i, 0))
_w_spec = pl.BlockSpec((D, D), lambda i: (0, 0))
_b_spec = pl.BlockSpec((1, D), lambda i: (0, 0))
_agg_spec = pl.BlockSpec((NC, R, H), lambda i: (0, i, 0))
_cnt_spec = pl.BlockSpec((R, 16), lambda i: (i, 0))

_u_shapes = [jax.ShapeDtypeStruct((N, H), f32),
             jax.ShapeDtypeStruct((N, H), f32),
             jax.ShapeDtypeStruct((N, D), f32)]

_tc_first = pl.pallas_call(
    _tc_first_body,
    grid=(G,),
    in_specs=[_row_spec, _w_spec, _w_spec, _b_spec],
    out_specs=[_half_spec, _half_spec, _row_spec],
    out_shape=_u_shapes,
)

_tc_mid = pl.pallas_call(
    _tc_mid_body,
    grid=(G,),
    in_specs=[_agg_spec, _cnt_spec, _row_spec, _w_spec, _w_spec, _b_spec],
    out_specs=[_half_spec, _half_spec, _row_spec],
    out_shape=_u_shapes,
)

_tc_last = pl.pallas_call(
    _tc_last_body,
    grid=(G,),
    in_specs=[_agg_spec, _cnt_spec, _row_spec],
    out_specs=_row_spec,
    out_shape=jax.ShapeDtypeStruct((N, D), f32),
)


@jax.jit
def kernel(x, edge_index, Wl1, bl1, Wr1, Wl2, bl2, Wr2, Wl3, bl3, Wr3):
    src = edge_index[0].reshape(EP // B, B)
    dst = edge_index[1].reshape(EP // B, B)

    ua1, ub1, v1 = _tc_first(x, Wl1, Wr1, bl1.reshape(1, D))
    agg1, cnt = _sc_agg_cnt(ua1, ub1, src, dst)
    ua2, ub2, v2 = _tc_mid(agg1, cnt, v1, Wl2, Wr2, bl2.reshape(1, D))
    agg2 = _sc_agg(ua2, ub2, src, dst)
    ua3, ub3, v3 = _tc_mid(agg2, cnt, v2, Wl3, Wr3, bl3.reshape(1, D))
    agg3 = _sc_agg(ua3, ub3, src, dst)
    return _tc_last(agg3, cnt, v3)


# speedup vs baseline: 2.5514x; 1.0199x over previous
import jax
import jax.numpy as jnp
from jax import lax
from jax.experimental import pallas as pl
from jax.experimental.pallas import tpu as pltpu
from jax.experimental.pallas import tpu_sc as plsc

N = 10000
E = 320000
D = 128
H = D // 2

NC = 2
NS = 16
B = 80
EP = E
EPT = EP // NS
NBLK = EPT // B
NP = 10240
RPS = NP // NS
ZR = 32

f32 = jnp.float32

_mesh = plsc.VectorSubcoreMesh(core_axis_name="c", subcore_axis_name="s")


def _zero_vmem(buf, rows, width):
    def zrow(i, carry):
        for j in range(width // 16):
            buf[i, pl.ds(j * 16, 16)] = jnp.zeros((16,), f32)
        return carry
    lax.fori_loop(0, rows, zrow, 0)


def _edge_loop(u_hbm, src_i, dst_i, bufs, agg_sh, sems_g, sems_s, lead,
               cnt_sh=None, ones_v=None, sem_c=None):
    nslot = len(bufs)
    for k in range(lead):
        pltpu.async_copy(u_hbm.at[src_i.at[k]], bufs[k], sems_g[k])

    dwait = nslot - lead

    def grp(g, carry):
        t0 = g * nslot
        for k in range(nslot):
            t = t0 + k
            buf, sem_g, sem_s = bufs[k], sems_g[k], sems_s[k]
            pltpu.make_async_copy(u_hbm.at[src_i.at[t]], buf, sem_g).wait()
            pltpu.async_copy(buf, agg_sh.at[dst_i.at[t]], sem_s, add=True)
            if cnt_sh is not None:
                pltpu.async_copy(ones_v, cnt_sh.at[dst_i.at[t]], sem_c,
                                 add=True)

            kw = (k - dwait) % nslot

            @pl.when(t >= dwait)
            def _():
                pltpu.make_async_copy(bufs[kw], agg_sh.at[dst_i.at[t]],
                                      sems_s[kw]).wait()

            kn = (k + lead) % nslot

            @pl.when(t + lead < NBLK)
            def _():
                pltpu.async_copy(u_hbm.at[src_i.at[t + lead]], bufs[kn],
                                 sems_g[kn])
        return carry
    lax.fori_loop(0, NBLK // nslot, grp, 0)

    for t in range(NBLK - dwait, NBLK):
        k = t % nslot
        pltpu.make_async_copy(bufs[k], agg_sh.at[dst_i.at[0]],
                              sems_s[k]).wait()

    if cnt_sh is not None:
        def drain(i, carry):
            pltpu.make_async_copy(ones_v, cnt_sh.at[dst_i.at[0]],
                                  sem_c).wait()
            return carry
        lax.fori_loop(0, NBLK, drain, 0)


def _sc_agg_cnt_body(ua_hbm, ub_hbm, edges_hbm, agg_out, cnt_out,
                     src_i, dst_i, b0, b1, b2, b3, b4,
                     zbuf, zcnt, ones_v,
                     agg_sh, cnt_sh, g0, g1, g2, g3, g4,
                     s0, s1, s2, s3, s4, sem_c):
    bufs = (b0, b1, b2, b3, b4)
    sems_g = (g0, g1, g2, g3, g4)
    sems_s = (s0, s1, s2, s3, s4)
    c = lax.axis_index("c")
    s = lax.axis_index("s")

    pltpu.sync_copy(edges_hbm.at[0, pl.ds(s * NBLK, NBLK)], src_i)
    pltpu.sync_copy(edges_hbm.at[1, pl.ds(s * NBLK, NBLK)], dst_i)

    _zero_vmem(zbuf, ZR, H)
    for t in range(RPS // ZR):
        pltpu.sync_copy(zbuf, agg_sh.at[pl.ds(s * RPS + t * ZR, ZR)])

    @pl.when(c == 0)
    def _():
        _zero_vmem(zcnt, ZR, 16)

        def orow(i, carry):
            ones_v[i, :] = jnp.ones((16,), f32)
            return carry
        lax.fori_loop(0, B, orow, 0)
        for t in range(RPS // ZR):
            pltpu.sync_copy(zcnt, cnt_sh.at[pl.ds(s * RPS + t * ZR, ZR)])

    plsc.subcore_barrier()

    @pl.when(c == 0)
    def _():
        _edge_loop(ua_hbm, src_i, dst_i, bufs, agg_sh, sems_g, sems_s, 4,
                   cnt_sh, ones_v, sem_c)

    @pl.when(c == 1)
    def _():
        _edge_loop(ub_hbm, src_i, dst_i, bufs, agg_sh, sems_g, sems_s, 4)

    plsc.subcore_barrier()

    pltpu.sync_copy(agg_sh.at[pl.ds(s * RPS, RPS)],
                    agg_out.at[c, pl.ds(s * RPS, RPS)])

    @pl.when(c == 0)
    def _():
        pltpu.sync_copy(cnt_sh.at[pl.ds(s * RPS, RPS)],
                        cnt_out.at[pl.ds(s * RPS, RPS)])


def _sc_agg_body(ua_hbm, ub_hbm, edges_hbm, agg_out,
                 src_i, dst_i, b0, b1, b2, b3, b4,
                 zbuf, agg_sh,
                 g0, g1, g2, g3, g4,
                 s0, s1, s2, s3, s4):
    bufs = (b0, b1, b2, b3, b4)
    sems_g = (g0, g1, g2, g3, g4)
    sems_s = (s0, s1, s2, s3, s4)
    c = lax.axis_index("c")
    s = lax.axis_index("s")

    pltpu.sync_copy(edges_hbm.at[0, pl.ds(s * NBLK, NBLK)], src_i)
    pltpu.sync_copy(edges_hbm.at[1, pl.ds(s * NBLK, NBLK)], dst_i)

    _zero_vmem(zbuf, ZR, H)
    for t in range(RPS // ZR):
        pltpu.sync_copy(zbuf, agg_sh.at[pl.ds(s * RPS + t * ZR, ZR)])

    plsc.subcore_barrier()

    @pl.when(c == 0)
    def _():
        _edge_loop(ua_hbm, src_i, dst_i, bufs, agg_sh, sems_g, sems_s, 4)

    @pl.when(c == 1)
    def _():
        _edge_loop(ub_hbm, src_i, dst_i, bufs, agg_sh, sems_g, sems_s, 4)

    plsc.subcore_barrier()

    pltpu.sync_copy(agg_sh.at[pl.ds(s * RPS, RPS)],
                    agg_out.at[c, pl.ds(s * RPS, RPS)])


_sc_agg_cnt = pl.kernel(
    _sc_agg_cnt_body,
    out_type=(jax.ShapeDtypeStruct((NC, NP, H), f32),
              jax.ShapeDtypeStruct((NP, 16), f32)),
    mesh=_mesh,
    compiler_params=pltpu.CompilerParams(use_tc_tiling_on_sc=False),
    scratch_types=[
        pltpu.VMEM((NBLK, B), jnp.int32),
        pltpu.VMEM((NBLK, B), jnp.int32),
        pltpu.VMEM((B, H), f32),
        pltpu.VMEM((B, H), f32),
        pltpu.VMEM((B, H), f32),
        pltpu.VMEM((B, H), f32),
        pltpu.VMEM((B, H), f32),
        pltpu.VMEM((ZR, H), f32),
        pltpu.VMEM((ZR, 16), f32),
        pltpu.VMEM((B, 16), f32),
        pltpu.VMEM_SHARED((NP, H), f32),
        pltpu.VMEM_SHARED((NP, 16), f32),
        pltpu.SemaphoreType.DMA,
        pltpu.SemaphoreType.DMA,
        pltpu.SemaphoreType.DMA,
        pltpu.SemaphoreType.DMA,
        pltpu.SemaphoreType.DMA,
        pltpu.SemaphoreType.DMA,
        pltpu.SemaphoreType.DMA,
        pltpu.SemaphoreType.DMA,
        pltpu.SemaphoreType.DMA,
        pltpu.SemaphoreType.DMA,
        pltpu.SemaphoreType.DMA,
    ],
)

_sc_agg = pl.kernel(
    _sc_agg_body,
    out_type=jax.ShapeDtypeStruct((NC, NP, H), f32),
    mesh=_mesh,
    compiler_params=pltpu.CompilerParams(use_tc_tiling_on_sc=False),
    scratch_types=[
        pltpu.VMEM((NBLK, B), jnp.int32),
        pltpu.VMEM((NBLK, B), jnp.int32),
        pltpu.VMEM((B, H), f32),
        pltpu.VMEM((B, H), f32),
        pltpu.VMEM((B, H), f32),
        pltpu.VMEM((B, H), f32),
        pltpu.VMEM((B, H), f32),
        pltpu.VMEM((ZR, H), f32),
        pltpu.VMEM_SHARED((NP, H), f32),
        pltpu.SemaphoreType.DMA,
        pltpu.SemaphoreType.DMA,
        pltpu.SemaphoreType.DMA,
        pltpu.SemaphoreType.DMA,
        pltpu.SemaphoreType.DMA,
        pltpu.SemaphoreType.DMA,
        pltpu.SemaphoreType.DMA,
        pltpu.SemaphoreType.DMA,
        pltpu.SemaphoreType.DMA,
        pltpu.SemaphoreType.DMA,
    ],
)


R = 1000
G = N // R

_DOT = (((1,), (1,)), ((), ()))


def _write_u_halves(u, ua_ref, ub_ref):
    ua_ref[...] = u[:, :H]
    ub_ref[...] = u[:, H:]


def _tc_first_body(x_ref, w_ref, bl_ref, ua_ref, ub_ref, v_ref):
    h = x_ref[...]
    uv = lax.dot_general(h, w_ref[...], _DOT, preferred_element_type=f32)
    _write_u_halves(uv[:, :D], ua_ref, ub_ref)
    v_ref[...] = uv[:, D:] + bl_ref[...]


def _tc_mid_body(agg_ref, cnt_ref, vp_ref, w_ref, bl_ref,
                 ua_ref, ub_ref, v_ref):
    inv = 1.0 / jnp.maximum(cnt_ref[:, 0:1], 1.0)
    mean = jnp.concatenate([agg_ref[0], agg_ref[1]], axis=1) * inv
    h = jnp.maximum(mean + vp_ref[...], 0.0)
    uv = lax.dot_general(h, w_ref[...], _DOT, preferred_element_type=f32)
    _write_u_halves(uv[:, :D], ua_ref, ub_ref)
    v_ref[...] = uv[:, D:] + bl_ref[...]


def _tc_last_body(agg_ref, cnt_ref, vp_ref, out_ref):
    inv = 1.0 / jnp.maximum(cnt_ref[:, 0:1], 1.0)
    mean = jnp.concatenate([agg_ref[0], agg_ref[1]], axis=1) * inv
    out_ref[...] = mean + vp_ref[...]


_row_spec = pl.BlockSpec((R, D), lambda i: (i, 0))
_half_spec = pl.BlockSpec((R, H), lambda i: (i, 0))
_w_spec = pl.BlockSpec((2 * D, D), lambda i: (0, 0))
_b_spec = pl.BlockSpec((1, D), lambda i: (0, 0))
_agg_spec = pl.BlockSpec((NC, R, H), lambda i: (0, i, 0))
_cnt_spec = pl.BlockSpec((R, 16), lambda i: (i, 0))

_u_shapes = [jax.ShapeDtypeStruct((N, H), f32),
             jax.ShapeDtypeStruct((N, H), f32),
             jax.ShapeDtypeStruct((N, D), f32)]

_tc_first = pl.pallas_call(
    _tc_first_body,
    grid=(G,),
    in_specs=[_row_spec, _w_spec, _b_spec],
    out_specs=[_half_spec, _half_spec, _row_spec],
    out_shape=_u_shapes,
)

_tc_mid = pl.pallas_call(
    _tc_mid_body,
    grid=(G,),
    in_specs=[_agg_spec, _cnt_spec, _row_spec, _w_spec, _b_spec],
    out_specs=[_half_spec, _half_spec, _row_spec],
    out_shape=_u_shapes,
)

_tc_last = pl.pallas_call(
    _tc_last_body,
    grid=(G,),
    in_specs=[_agg_spec, _cnt_spec, _row_spec],
    out_specs=_row_spec,
    out_shape=jax.ShapeDtypeStruct((N, D), f32),
)


@jax.jit
def kernel(x, edge_index, Wl1, bl1, Wr1, Wl2, bl2, Wr2, Wl3, bl3, Wr3):
    edges = edge_index.reshape(2, EP // B, B)
    w1 = jnp.concatenate([Wl1, Wr1], axis=0)
    w2 = jnp.concatenate([Wl2, Wr2], axis=0)
    w3 = jnp.concatenate([Wl3, Wr3], axis=0)

    ua1, ub1, v1 = _tc_first(x, w1, bl1.reshape(1, D))
    agg1, cnt = _sc_agg_cnt(ua1, ub1, edges)
    ua2, ub2, v2 = _tc_mid(agg1, cnt, v1, w2, bl2.reshape(1, D))
    agg2 = _sc_agg(ua2, ub2, edges)
    ua3, ub3, v3 = _tc_mid(agg2, cnt, v2, w3, bl3.reshape(1, D))
    agg3 = _sc_agg(ua3, ub3, edges)
    return _tc_last(agg3, cnt, v3)


# speedup vs baseline: 2.5938x; 1.0166x over previous
import jax
import jax.numpy as jnp
from jax import lax
from jax.experimental import pallas as pl
from jax.experimental.pallas import tpu as pltpu
from jax.experimental.pallas import tpu_sc as plsc

N = 10000
E = 320000
D = 128
H = D // 2

NC = 2
NS = 16
B = 80
EP = E
EPT = EP // NS
NBLK = EPT // B
NP = 10240
RPS = NP // NS
ZR = 32

f32 = jnp.float32

_mesh = plsc.VectorSubcoreMesh(core_axis_name="c", subcore_axis_name="s")


def _zero_vmem(buf, rows, width):
    def zrow(i, carry):
        for j in range(width // 16):
            buf[i, pl.ds(j * 16, 16)] = jnp.zeros((16,), f32)
        return carry
    lax.fori_loop(0, rows, zrow, 0)


def _edge_loop(u_hbm, src_i, dst_i, bufs, agg_sh, sems_g, sems_s, lead,
               cnt_sh=None, ones_v=None, sem_c=None):
    nslot = len(bufs)
    for k in range(lead):
        pltpu.async_copy(u_hbm.at[src_i.at[k]], bufs[k], sems_g[k])

    dwait = nslot - lead

    def grp(g, carry):
        t0 = g * nslot
        for k in range(nslot):
            t = t0 + k
            buf, sem_g, sem_s = bufs[k], sems_g[k], sems_s[k]
            pltpu.make_async_copy(u_hbm.at[src_i.at[t]], buf, sem_g).wait()
            pltpu.async_copy(buf, agg_sh.at[dst_i.at[t]], sem_s, add=True)
            if cnt_sh is not None:
                pltpu.async_copy(ones_v, cnt_sh.at[dst_i.at[t]], sem_c,
                                 add=True)

            kw = (k - dwait) % nslot

            @pl.when(t >= dwait)
            def _():
                pltpu.make_async_copy(bufs[kw], agg_sh.at[dst_i.at[t]],
                                      sems_s[kw]).wait()

            kn = (k + lead) % nslot

            @pl.when(t + lead < NBLK)
            def _():
                pltpu.async_copy(u_hbm.at[src_i.at[t + lead]], bufs[kn],
                                 sems_g[kn])
        return carry
    lax.fori_loop(0, NBLK // nslot, grp, 0)

    for t in range(NBLK - dwait, NBLK):
        k = t % nslot
        pltpu.make_async_copy(bufs[k], agg_sh.at[dst_i.at[0]],
                              sems_s[k]).wait()

    if cnt_sh is not None:
        def drain(i, carry):
            pltpu.make_async_copy(ones_v, cnt_sh.at[dst_i.at[0]],
                                  sem_c).wait()
            return carry
        lax.fori_loop(0, NBLK, drain, 0)


def _sc_agg_cnt_body(ua_hbm, ub_hbm, edges_hbm, agg_out, cnt_out,
                     src_i, dst_i, b0, b1, b2, b3, b4,
                     zbuf, zcnt, ones_v,
                     agg_sh, cnt_sh, g0, g1, g2, g3, g4,
                     s0, s1, s2, s3, s4, sem_c):
    bufs = (b0, b1, b2, b3, b4)
    sems_g = (g0, g1, g2, g3, g4)
    sems_s = (s0, s1, s2, s3, s4)
    c = lax.axis_index("c")
    s = lax.axis_index("s")

    pltpu.sync_copy(edges_hbm.at[0, pl.ds(s * NBLK, NBLK)], src_i)
    pltpu.sync_copy(edges_hbm.at[1, pl.ds(s * NBLK, NBLK)], dst_i)

    _zero_vmem(zbuf, ZR, H)
    for t in range(RPS // ZR):
        pltpu.sync_copy(zbuf, agg_sh.at[pl.ds(s * RPS + t * ZR, ZR)])

    @pl.when(c == 0)
    def _():
        _zero_vmem(zcnt, ZR, 16)

        def orow(i, carry):
            ones_v[i, :] = jnp.ones((16,), f32)
            return carry
        lax.fori_loop(0, B, orow, 0)
        for t in range(RPS // ZR):
            pltpu.sync_copy(zcnt, cnt_sh.at[pl.ds(s * RPS + t * ZR, ZR)])

    plsc.subcore_barrier()

    @pl.when(c == 0)
    def _():
        _edge_loop(ua_hbm, src_i, dst_i, bufs, agg_sh, sems_g, sems_s, 4,
                   cnt_sh, ones_v, sem_c)

    @pl.when(c == 1)
    def _():
        _edge_loop(ub_hbm, src_i, dst_i, bufs, agg_sh, sems_g, sems_s, 4)

    plsc.subcore_barrier()

    pltpu.sync_copy(agg_sh.at[pl.ds(s * RPS, RPS)],
                    agg_out.at[c, pl.ds(s * RPS, RPS)])

    @pl.when(c == 0)
    def _():
        pltpu.sync_copy(cnt_sh.at[pl.ds(s * RPS, RPS)],
                        cnt_out.at[pl.ds(s * RPS, RPS)])


def _sc_agg_body(ua_hbm, ub_hbm, edges_hbm, agg_out,
                 src_i, dst_i, b0, b1, b2, b3, b4,
                 zbuf, agg_sh,
                 g0, g1, g2, g3, g4,
                 s0, s1, s2, s3, s4):
    bufs = (b0, b1, b2, b3, b4)
    sems_g = (g0, g1, g2, g3, g4)
    sems_s = (s0, s1, s2, s3, s4)
    c = lax.axis_index("c")
    s = lax.axis_index("s")

    pltpu.sync_copy(edges_hbm.at[0, pl.ds(s * NBLK, NBLK)], src_i)
    pltpu.sync_copy(edges_hbm.at[1, pl.ds(s * NBLK, NBLK)], dst_i)

    _zero_vmem(zbuf, ZR, H)
    for t in range(RPS // ZR):
        pltpu.sync_copy(zbuf, agg_sh.at[pl.ds(s * RPS + t * ZR, ZR)])

    plsc.subcore_barrier()

    @pl.when(c == 0)
    def _():
        _edge_loop(ua_hbm, src_i, dst_i, bufs, agg_sh, sems_g, sems_s, 4)

    @pl.when(c == 1)
    def _():
        _edge_loop(ub_hbm, src_i, dst_i, bufs, agg_sh, sems_g, sems_s, 4)

    plsc.subcore_barrier()

    pltpu.sync_copy(agg_sh.at[pl.ds(s * RPS, RPS)],
                    agg_out.at[c, pl.ds(s * RPS, RPS)])


_sc_agg_cnt = pl.kernel(
    _sc_agg_cnt_body,
    out_type=(jax.ShapeDtypeStruct((NC, NP, H), f32),
              jax.ShapeDtypeStruct((NP, 16), f32)),
    mesh=_mesh,
    compiler_params=pltpu.CompilerParams(use_tc_tiling_on_sc=False),
    scratch_types=[
        pltpu.VMEM((NBLK, B), jnp.int32),
        pltpu.VMEM((NBLK, B), jnp.int32),
        pltpu.VMEM((B, H), f32),
        pltpu.VMEM((B, H), f32),
        pltpu.VMEM((B, H), f32),
        pltpu.VMEM((B, H), f32),
        pltpu.VMEM((B, H), f32),
        pltpu.VMEM((ZR, H), f32),
        pltpu.VMEM((ZR, 16), f32),
        pltpu.VMEM((B, 16), f32),
        pltpu.VMEM_SHARED((NP, H), f32),
        pltpu.VMEM_SHARED((NP, 16), f32),
        pltpu.SemaphoreType.DMA,
        pltpu.SemaphoreType.DMA,
        pltpu.SemaphoreType.DMA,
        pltpu.SemaphoreType.DMA,
        pltpu.SemaphoreType.DMA,
        pltpu.SemaphoreType.DMA,
        pltpu.SemaphoreType.DMA,
        pltpu.SemaphoreType.DMA,
        pltpu.SemaphoreType.DMA,
        pltpu.SemaphoreType.DMA,
        pltpu.SemaphoreType.DMA,
    ],
)

_sc_agg = pl.kernel(
    _sc_agg_body,
    out_type=jax.ShapeDtypeStruct((NC, NP, H), f32),
    mesh=_mesh,
    compiler_params=pltpu.CompilerParams(use_tc_tiling_on_sc=False),
    scratch_types=[
        pltpu.VMEM((NBLK, B), jnp.int32),
        pltpu.VMEM((NBLK, B), jnp.int32),
        pltpu.VMEM((B, H), f32),
        pltpu.VMEM((B, H), f32),
        pltpu.VMEM((B, H), f32),
        pltpu.VMEM((B, H), f32),
        pltpu.VMEM((B, H), f32),
        pltpu.VMEM((ZR, H), f32),
        pltpu.VMEM_SHARED((NP, H), f32),
        pltpu.SemaphoreType.DMA,
        pltpu.SemaphoreType.DMA,
        pltpu.SemaphoreType.DMA,
        pltpu.SemaphoreType.DMA,
        pltpu.SemaphoreType.DMA,
        pltpu.SemaphoreType.DMA,
        pltpu.SemaphoreType.DMA,
        pltpu.SemaphoreType.DMA,
        pltpu.SemaphoreType.DMA,
        pltpu.SemaphoreType.DMA,
    ],
)


R = 2000
G = N // R

_DOT = (((1,), (1,)), ((), ()))


def _write_u_halves(u, ua_ref, ub_ref):
    ua_ref[...] = u[:, :H]
    ub_ref[...] = u[:, H:]


def _tc_first_body(x_ref, w_ref, bl_ref, ua_ref, ub_ref, v_ref):
    h = x_ref[...]
    uv = lax.dot_general(h, w_ref[...], _DOT, preferred_element_type=f32)
    _write_u_halves(uv[:, :D], ua_ref, ub_ref)
    v_ref[...] = uv[:, D:] + bl_ref[...]


def _tc_mid_body(agg_ref, cnt_ref, vp_ref, w_ref, bl_ref,
                 ua_ref, ub_ref, v_ref):
    inv = 1.0 / jnp.maximum(cnt_ref[:, 0:1], 1.0)
    mean = jnp.concatenate([agg_ref[0], agg_ref[1]], axis=1) * inv
    h = jnp.maximum(mean + vp_ref[...], 0.0)
    uv = lax.dot_general(h, w_ref[...], _DOT, preferred_element_type=f32)
    _write_u_halves(uv[:, :D], ua_ref, ub_ref)
    v_ref[...] = uv[:, D:] + bl_ref[...]


def _tc_last_body(agg_ref, cnt_ref, vp_ref, out_ref):
    inv = 1.0 / jnp.maximum(cnt_ref[:, 0:1], 1.0)
    mean = jnp.concatenate([agg_ref[0], agg_ref[1]], axis=1) * inv
    out_ref[...] = mean + vp_ref[...]


_row_spec = pl.BlockSpec((R, D), lambda i: (i, 0))
_half_spec = pl.BlockSpec((R, H), lambda i: (i, 0))
_w_spec = pl.BlockSpec((2 * D, D), lambda i: (0, 0))
_b_spec = pl.BlockSpec((1, D), lambda i: (0, 0))
_agg_spec = pl.BlockSpec((NC, R, H), lambda i: (0, i, 0))
_cnt_spec = pl.BlockSpec((R, 16), lambda i: (i, 0))

_u_shapes = [jax.ShapeDtypeStruct((N, H), f32),
             jax.ShapeDtypeStruct((N, H), f32),
             jax.ShapeDtypeStruct((N, D), f32)]

_tc_first = pl.pallas_call(
    _tc_first_body,
    grid=(G,),
    in_specs=[_row_spec, _w_spec, _b_spec],
    out_specs=[_half_spec, _half_spec, _row_spec],
    out_shape=_u_shapes,
)

_tc_mid = pl.pallas_call(
    _tc_mid_body,
    grid=(G,),
    in_specs=[_agg_spec, _cnt_spec, _row_spec, _w_spec, _b_spec],
    out_specs=[_half_spec, _half_spec, _row_spec],
    out_shape=_u_shapes,
)

_tc_last = pl.pallas_call(
    _tc_last_body,
    grid=(G,),
    in_specs=[_agg_spec, _cnt_spec, _row_spec],
    out_specs=_row_spec,
    out_shape=jax.ShapeDtypeStruct((N, D), f32),
)


@jax.jit
def kernel(x, edge_index, Wl1, bl1, Wr1, Wl2, bl2, Wr2, Wl3, bl3, Wr3):
    edges = edge_index.reshape(2, EP // B, B)
    w1 = jnp.concatenate([Wl1, Wr1], axis=0)
    w2 = jnp.concatenate([Wl2, Wr2], axis=0)
    w3 = jnp.concatenate([Wl3, Wr3], axis=0)

    ua1, ub1, v1 = _tc_first(x, w1, bl1.reshape(1, D))
    agg1, cnt = _sc_agg_cnt(ua1, ub1, edges)
    ua2, ub2, v2 = _tc_mid(agg1, cnt, v1, w2, bl2.reshape(1, D))
    agg2 = _sc_agg(ua2, ub2, edges)
    ua3, ub3, v3 = _tc_mid(agg2, cnt, v2, w3, bl3.reshape(1, D))
    agg3 = _sc_agg(ua3, ub3, edges)
    return _tc_last(agg3, cnt, v3)


# speedup vs baseline: 3.3086x; 1.2756x over previous
import jax
import jax.numpy as jnp
from jax import lax
from jax.experimental import pallas as pl
from jax.experimental.pallas import tpu as pltpu
from jax.experimental.pallas import tpu_sc as plsc

N = 10000
E = 320000
D = 128
H = D // 2

NC = 2
NS = 16
B = 80
EP = E
EPT = EP // NS
NBLK = EPT // B
NP = 10240
RPS = NP // NS
ZR = 32

f32 = jnp.float32
bf16 = jnp.bfloat16

_mesh = plsc.VectorSubcoreMesh(core_axis_name="c", subcore_axis_name="s")


def _zero_vmem(buf, rows, width, dtype=f32):
    lanes = 32 if dtype == bf16 else 16
    def zrow(i, carry):
        for j in range(width // lanes):
            buf[i, pl.ds(j * lanes, lanes)] = jnp.zeros((lanes,), dtype)
        return carry
    lax.fori_loop(0, rows, zrow, 0)


def _edge_loop(u_hbm, src_i, dst_i, bufs, agg_sh, sems_g, sems_s, lead,
               cnt_sh=None, ones_v=None, sem_c=None):
    nslot = len(bufs)
    for k in range(lead):
        pltpu.async_copy(u_hbm.at[src_i.at[k]], bufs[k], sems_g[k])

    dwait = nslot - lead

    def grp(g, carry):
        t0 = g * nslot
        for k in range(nslot):
            t = t0 + k
            buf, sem_g, sem_s = bufs[k], sems_g[k], sems_s[k]
            pltpu.make_async_copy(u_hbm.at[src_i.at[t]], buf, sem_g).wait()
            pltpu.async_copy(buf, agg_sh.at[dst_i.at[t]], sem_s, add=True)
            if cnt_sh is not None:
                pltpu.async_copy(ones_v, cnt_sh.at[dst_i.at[t]], sem_c,
                                 add=True)

            kw = (k - dwait) % nslot

            @pl.when(t >= dwait)
            def _():
                pltpu.make_async_copy(bufs[kw], agg_sh.at[dst_i.at[t]],
                                      sems_s[kw]).wait()

            kn = (k + lead) % nslot

            @pl.when(t + lead < NBLK)
            def _():
                pltpu.async_copy(u_hbm.at[src_i.at[t + lead]], bufs[kn],
                                 sems_g[kn])
        return carry
    lax.fori_loop(0, NBLK // nslot, grp, 0)

    for t in range(NBLK - dwait, NBLK):
        k = t % nslot
        pltpu.make_async_copy(bufs[k], agg_sh.at[dst_i.at[0]],
                              sems_s[k]).wait()

    if cnt_sh is not None:
        def drain(i, carry):
            pltpu.make_async_copy(ones_v, cnt_sh.at[dst_i.at[0]],
                                  sem_c).wait()
            return carry
        lax.fori_loop(0, NBLK, drain, 0)


def _sc_agg_cnt_body(ua_hbm, ub_hbm, edges_hbm, agg_out, cnt_out,
                     src_i, dst_i, b0, b1, b2, b3, b4,
                     zbuf, zcnt, ones_v,
                     agg_sh, cnt_sh, g0, g1, g2, g3, g4,
                     s0, s1, s2, s3, s4, sem_c):
    bufs = (b0, b1, b2, b3, b4)
    sems_g = (g0, g1, g2, g3, g4)
    sems_s = (s0, s1, s2, s3, s4)
    c = lax.axis_index("c")
    s = lax.axis_index("s")

    pltpu.sync_copy(edges_hbm.at[0, pl.ds(s * NBLK, NBLK)], src_i)
    pltpu.sync_copy(edges_hbm.at[1, pl.ds(s * NBLK, NBLK)], dst_i)

    _zero_vmem(zbuf, ZR, H, bf16)
    for t in range(RPS // ZR):
        pltpu.sync_copy(zbuf, agg_sh.at[pl.ds(s * RPS + t * ZR, ZR)])

    @pl.when(c == 0)
    def _():
        _zero_vmem(zcnt, ZR, 16)

        def orow(i, carry):
            ones_v[i, :] = jnp.ones((16,), f32)
            return carry
        lax.fori_loop(0, B, orow, 0)
        for t in range(RPS // ZR):
            pltpu.sync_copy(zcnt, cnt_sh.at[pl.ds(s * RPS + t * ZR, ZR)])

    plsc.subcore_barrier()

    @pl.when(c == 0)
    def _():
        _edge_loop(ua_hbm, src_i, dst_i, bufs, agg_sh, sems_g, sems_s, 4,
                   cnt_sh, ones_v, sem_c)

    @pl.when(c == 1)
    def _():
        _edge_loop(ub_hbm, src_i, dst_i, bufs, agg_sh, sems_g, sems_s, 4)

    plsc.subcore_barrier()

    pltpu.sync_copy(agg_sh.at[pl.ds(s * RPS, RPS)],
                    agg_out.at[c, pl.ds(s * RPS, RPS)])

    @pl.when(c == 0)
    def _():
        pltpu.sync_copy(cnt_sh.at[pl.ds(s * RPS, RPS)],
                        cnt_out.at[pl.ds(s * RPS, RPS)])


def _sc_agg_body(ua_hbm, ub_hbm, edges_hbm, agg_out,
                 src_i, dst_i, b0, b1, b2, b3, b4,
                 zbuf, agg_sh,
                 g0, g1, g2, g3, g4,
                 s0, s1, s2, s3, s4):
    bufs = (b0, b1, b2, b3, b4)
    sems_g = (g0, g1, g2, g3, g4)
    sems_s = (s0, s1, s2, s3, s4)
    c = lax.axis_index("c")
    s = lax.axis_index("s")

    pltpu.sync_copy(edges_hbm.at[0, pl.ds(s * NBLK, NBLK)], src_i)
    pltpu.sync_copy(edges_hbm.at[1, pl.ds(s * NBLK, NBLK)], dst_i)

    _zero_vmem(zbuf, ZR, H, bf16)
    for t in range(RPS // ZR):
        pltpu.sync_copy(zbuf, agg_sh.at[pl.ds(s * RPS + t * ZR, ZR)])

    plsc.subcore_barrier()

    @pl.when(c == 0)
    def _():
        _edge_loop(ua_hbm, src_i, dst_i, bufs, agg_sh, sems_g, sems_s, 4)

    @pl.when(c == 1)
    def _():
        _edge_loop(ub_hbm, src_i, dst_i, bufs, agg_sh, sems_g, sems_s, 4)

    plsc.subcore_barrier()

    pltpu.sync_copy(agg_sh.at[pl.ds(s * RPS, RPS)],
                    agg_out.at[c, pl.ds(s * RPS, RPS)])


_sc_agg_cnt = pl.kernel(
    _sc_agg_cnt_body,
    out_type=(jax.ShapeDtypeStruct((NC, NP, H), bf16),
              jax.ShapeDtypeStruct((NP, 16), f32)),
    mesh=_mesh,
    compiler_params=pltpu.CompilerParams(use_tc_tiling_on_sc=False),
    scratch_types=[
        pltpu.VMEM((NBLK, B), jnp.int32),
        pltpu.VMEM((NBLK, B), jnp.int32),
        pltpu.VMEM((B, H), bf16),
        pltpu.VMEM((B, H), bf16),
        pltpu.VMEM((B, H), bf16),
        pltpu.VMEM((B, H), bf16),
        pltpu.VMEM((B, H), bf16),
        pltpu.VMEM((ZR, H), bf16),
        pltpu.VMEM((ZR, 16), f32),
        pltpu.VMEM((B, 16), f32),
        pltpu.VMEM_SHARED((NP, H), bf16),
        pltpu.VMEM_SHARED((NP, 16), f32),
        pltpu.SemaphoreType.DMA,
        pltpu.SemaphoreType.DMA,
        pltpu.SemaphoreType.DMA,
        pltpu.SemaphoreType.DMA,
        pltpu.SemaphoreType.DMA,
        pltpu.SemaphoreType.DMA,
        pltpu.SemaphoreType.DMA,
        pltpu.SemaphoreType.DMA,
        pltpu.SemaphoreType.DMA,
        pltpu.SemaphoreType.DMA,
        pltpu.SemaphoreType.DMA,
    ],
)

_sc_agg = pl.kernel(
    _sc_agg_body,
    out_type=jax.ShapeDtypeStruct((NC, NP, H), bf16),
    mesh=_mesh,
    compiler_params=pltpu.CompilerParams(use_tc_tiling_on_sc=False),
    scratch_types=[
        pltpu.VMEM((NBLK, B), jnp.int32),
        pltpu.VMEM((NBLK, B), jnp.int32),
        pltpu.VMEM((B, H), bf16),
        pltpu.VMEM((B, H), bf16),
        pltpu.VMEM((B, H), bf16),
        pltpu.VMEM((B, H), bf16),
        pltpu.VMEM((B, H), bf16),
        pltpu.VMEM((ZR, H), bf16),
        pltpu.VMEM_SHARED((NP, H), bf16),
        pltpu.SemaphoreType.DMA,
        pltpu.SemaphoreType.DMA,
        pltpu.SemaphoreType.DMA,
        pltpu.SemaphoreType.DMA,
        pltpu.SemaphoreType.DMA,
        pltpu.SemaphoreType.DMA,
        pltpu.SemaphoreType.DMA,
        pltpu.SemaphoreType.DMA,
        pltpu.SemaphoreType.DMA,
        pltpu.SemaphoreType.DMA,
    ],
)


R = 2000
G = N // R

_DOT = (((1,), (1,)), ((), ()))


def _write_u_halves(u, ua_ref, ub_ref):
    ua_ref[...] = u[:, :H]
    ub_ref[...] = u[:, H:]


def _tc_first_body(x_ref, w_ref, bl_ref, ua_ref, ub_ref, v_ref):
    h = x_ref[...]
    uv = lax.dot_general(h, w_ref[...], _DOT, preferred_element_type=f32)
    _write_u_halves(uv[:, :D].astype(bf16), ua_ref, ub_ref)
    v_ref[...] = uv[:, D:] + bl_ref[...]


def _tc_mid_body(agg_ref, cnt_ref, vp_ref, w_ref, bl_ref,
                 ua_ref, ub_ref, v_ref):
    inv = 1.0 / jnp.maximum(cnt_ref[:, 0:1], 1.0)
    mean = jnp.concatenate([agg_ref[0], agg_ref[1]],
                           axis=1).astype(f32) * inv
    h = jnp.maximum(mean + vp_ref[...], 0.0)
    uv = lax.dot_general(h, w_ref[...], _DOT, preferred_element_type=f32)
    _write_u_halves(uv[:, :D].astype(bf16), ua_ref, ub_ref)
    v_ref[...] = uv[:, D:] + bl_ref[...]


def _tc_last_body(agg_ref, cnt_ref, vp_ref, out_ref):
    inv = 1.0 / jnp.maximum(cnt_ref[:, 0:1], 1.0)
    mean = jnp.concatenate([agg_ref[0], agg_ref[1]],
                           axis=1).astype(f32) * inv
    out_ref[...] = mean + vp_ref[...]


_row_spec = pl.BlockSpec((R, D), lambda i: (i, 0))
_half_spec = pl.BlockSpec((R, H), lambda i: (i, 0))
_w_spec = pl.BlockSpec((2 * D, D), lambda i: (0, 0))
_b_spec = pl.BlockSpec((1, D), lambda i: (0, 0))
_agg_spec = pl.BlockSpec((NC, R, H), lambda i: (0, i, 0))
_cnt_spec = pl.BlockSpec((R, 16), lambda i: (i, 0))

_u_shapes = [jax.ShapeDtypeStruct((N, H), bf16),
             jax.ShapeDtypeStruct((N, H), bf16),
             jax.ShapeDtypeStruct((N, D), f32)]

_tc_first = pl.pallas_call(
    _tc_first_body,
    grid=(G,),
    in_specs=[_row_spec, _w_spec, _b_spec],
    out_specs=[_half_spec, _half_spec, _row_spec],
    out_shape=_u_shapes,
)

_tc_mid = pl.pallas_call(
    _tc_mid_body,
    grid=(G,),
    in_specs=[_agg_spec, _cnt_spec, _row_spec, _w_spec, _b_spec],
    out_specs=[_half_spec, _half_spec, _row_spec],
    out_shape=_u_shapes,
)

_tc_last = pl.pallas_call(
    _tc_last_body,
    grid=(G,),
    in_specs=[_agg_spec, _cnt_spec, _row_spec],
    out_specs=_row_spec,
    out_shape=jax.ShapeDtypeStruct((N, D), f32),
)


@jax.jit
def kernel(x, edge_index, Wl1, bl1, Wr1, Wl2, bl2, Wr2, Wl3, bl3, Wr3):
    edges = edge_index.reshape(2, EP // B, B)
    w1 = jnp.concatenate([Wl1, Wr1], axis=0)
    w2 = jnp.concatenate([Wl2, Wr2], axis=0)
    w3 = jnp.concatenate([Wl3, Wr3], axis=0)

    ua1, ub1, v1 = _tc_first(x, w1, bl1.reshape(1, D))
    agg1, cnt = _sc_agg_cnt(ua1, ub1, edges)
    ua2, ub2, v2 = _tc_mid(agg1, cnt, v1, w2, bl2.reshape(1, D))
    agg2 = _sc_agg(ua2, ub2, edges)
    ua3, ub3, v3 = _tc_mid(agg2, cnt, v2, w3, bl3.reshape(1, D))
    agg3 = _sc_agg(ua3, ub3, edges)
    return _tc_last(agg3, cnt, v3)
